# baseline (speedup 1.0000x reference)
import functools

import jax
import jax.numpy as jnp
from jax import lax
from jax.experimental import pallas as pl
from jax.experimental.pallas import tpu as pltpu
from jax.experimental.pallas import tpu_sc as plsc

_C = 1000
_B = 16384
_NC = 2
_NS = 16
_NW = _NC * _NS
_RPW = _B // _NW
_CHUNK = 64
_NCHUNK = _RPW // _CHUNK
_L = 16
_GROUPS = _CHUNK // _L


def _onehot_body(labels_hbm, zeros_hbm, out_hbm, lbl_v, buf):
    cid = lax.axis_index("c")
    sid = lax.axis_index("s")
    wid = sid * _NC + cid
    base = wid * _RPW

    pltpu.sync_copy(labels_hbm.at[pl.ds(base, _RPW)], lbl_v)
    pltpu.sync_copy(zeros_hbm, buf)

    ones_v = jnp.ones((_L,), jnp.int32)
    zeros_v = jnp.zeros((_L,), jnp.int32)
    lane_v = lax.iota(jnp.int32, _L)

    for g in range(_NCHUNK):
        row0 = g * _CHUNK
        for j in range(_GROUPS):
            cols = lbl_v[pl.ds(row0 + j * _L, _L)]
            idx = (lane_v + (j * _L)) * _C + cols
            plsc.store_scatter(buf, [idx], ones_v)
        pltpu.sync_copy(
            buf, out_hbm.at[pl.ds(base * _C + row0 * _C, _CHUNK * _C)])
        for j in range(_GROUPS):
            cols = lbl_v[pl.ds(row0 + j * _L, _L)]
            idx = (lane_v + (j * _L)) * _C + cols
            plsc.store_scatter(buf, [idx], zeros_v)


@jax.jit
def kernel(labels):
    labels = labels.astype(jnp.int32)
    zeros_block = jnp.zeros((_CHUNK * _C,), jnp.int32)
    mesh = plsc.VectorSubcoreMesh(core_axis_name="c", subcore_axis_name="s")
    run = functools.partial(
        pl.kernel,
        out_type=jax.ShapeDtypeStruct((_B * _C,), jnp.int32),
        mesh=mesh,
        scratch_types=[
            pltpu.VMEM((_RPW,), jnp.int32),
            pltpu.VMEM((_CHUNK * _C,), jnp.int32),
        ],
        compiler_params=pltpu.CompilerParams(needs_layout_passes=False),
    )(_onehot_body)
    return run(labels, zeros_block).reshape(_B, _C)


# speedup vs baseline: 1.4419x; 1.4419x over previous
import functools

import jax
import jax.numpy as jnp
from jax import lax
from jax.experimental import pallas as pl
from jax.experimental.pallas import tpu as pltpu
from jax.experimental.pallas import tpu_sc as plsc

_C = 1000
_B = 16384
_NC = 2
_NS = 16
_NW = _NC * _NS
_RPW = _B // _NW
_CHUNK = 32
_NCHUNK = _RPW // _CHUNK
_L = 16
_GROUPS = _CHUNK // _L
_NBUF = 2


def _onehot_body(labels_hbm, zeros_hbm, out_hbm, lbl_v, buf0, buf1,
                 sem0, sem1):
    cid = lax.axis_index("c")
    sid = lax.axis_index("s")
    wid = sid * _NC + cid
    base = wid * _RPW

    bufs = [buf0, buf1]
    sems = [sem0, sem1]

    pltpu.sync_copy(labels_hbm.at[pl.ds(base, _RPW)], lbl_v)
    zfill = []
    for b in range(_NBUF):
        d = pltpu.make_async_copy(zeros_hbm, bufs[b], sems[b])
        d.start()
        zfill.append(d)

    ones_v = jnp.ones((_L,), jnp.int32)
    zeros_v = jnp.zeros((_L,), jnp.int32)
    lane_v = lax.iota(jnp.int32, _L)

    def scatter(g, buf, val):
        row0 = g * _CHUNK
        for j in range(_GROUPS):
            rows = lane_v + (j * _L)
            cols = lbl_v[pl.ds(row0 + j * _L, _L)]
            plsc.store_scatter(buf, [rows, cols], val)

    copies = [None] * _NCHUNK
    for g in range(_NCHUNK):
        b = g % _NBUF
        if g < _NBUF:
            zfill[b].wait()
        else:
            copies[g - _NBUF].wait()
            scatter(g - _NBUF, bufs[b], zeros_v)
        scatter(g, bufs[b], ones_v)
        d = pltpu.make_async_copy(
            bufs[b],
            out_hbm.at[pl.ds(base + g * _CHUNK, _CHUNK), :],
            sems[b],
        )
        d.start()
        copies[g] = d
    for g in range(_NCHUNK - _NBUF, _NCHUNK):
        copies[g].wait()


@jax.jit
def kernel(labels):
    labels = labels.astype(jnp.int32)
    zeros_block = jnp.zeros((_CHUNK, _C), jnp.int32)
    mesh = plsc.VectorSubcoreMesh(core_axis_name="c", subcore_axis_name="s")
    run = functools.partial(
        pl.kernel,
        out_type=jax.ShapeDtypeStruct((_B, _C), jnp.int32),
        mesh=mesh,
        scratch_types=[
            pltpu.VMEM((_RPW,), jnp.int32),
            pltpu.VMEM((_CHUNK, _C), jnp.int32),
            pltpu.VMEM((_CHUNK, _C), jnp.int32),
            pltpu.SemaphoreType.DMA,
            pltpu.SemaphoreType.DMA,
        ],
        compiler_params=pltpu.CompilerParams(needs_layout_passes=False),
    )(_onehot_body)
    return run(labels, zeros_block)
